# baseline (speedup 1.0000x reference)
import jax
import jax.numpy as jnp
from jax.experimental import pallas as pl
from jax.experimental.pallas import tpu as pltpu


def _fused_kernel(text_ref, adj_ref, Wn_ref, bn_ref, gamma_ref, beta_ref,
                  pn_ref, Wg_ref, bg_ref, pg_ref, out_ref,
                  abuf_ref, obuf_ref, in_sem, out_sem):
    B, L, D = text_ref.shape

    def adj_copy(b):
        return pltpu.make_async_copy(adj_ref.at[b], abuf_ref.at[b],
                                     in_sem.at[b])

    def out_copy(b):
        return pltpu.make_async_copy(obuf_ref.at[b], out_ref.at[b],
                                     out_sem.at[b])

    for b in range(B):
        adj_copy(b).start()

    x = text_ref[...].reshape(B * L, D)
    h = jnp.dot(x, Wn_ref[...], preferred_element_type=jnp.float32) + bn_ref[...]
    mean = jnp.mean(h, axis=0, keepdims=True)
    var = jnp.mean((h - mean) * (h - mean), axis=0, keepdims=True)
    h = (h - mean) * jax.lax.rsqrt(var + 1e-5) * gamma_ref[...] + beta_ref[...]
    pn = pn_ref[0, 0]
    tn = jnp.where(h >= 0, h, pn * h)

    xl = jnp.dot(tn, Wg_ref[...], preferred_element_type=jnp.float32)

    pg = pg_ref[0, 0]
    ones_col = jnp.ones((L, 1), dtype=jnp.float32)
    row = jax.lax.broadcasted_iota(jnp.int32, (L, L), 0)
    col = jax.lax.broadcasted_iota(jnp.int32, (L, L), 1)
    diag = (row == col)

    dn = (((0,), (0,)), ((), ()))
    for b in range(B):
        adj_copy(b).wait()
        A = jnp.where(diag, 1.0, abuf_ref[b].astype(jnp.float32))
        deg = jax.lax.dot_general(A, ones_col, dn,
                                  preferred_element_type=jnp.float32)
        dinv = jax.lax.rsqrt(deg)
        msg = xl[b * L:(b + 1) * L] * dinv
        agg = jax.lax.dot_general(A, msg, dn,
                                  preferred_element_type=jnp.float32)
        hid = agg * dinv + bg_ref[...]
        g = jnp.where(hid >= 0, hid, pg * hid)
        nrm = jnp.sqrt(jnp.sum(g * g, axis=1, keepdims=True))
        g = g / jnp.maximum(nrm, 1e-12)
        obuf_ref[b] = g + text_ref[b]
        out_copy(b).start()

    for b in range(B):
        out_copy(b).wait()


def kernel(text_feature, adj, W_node, b_node, bn_gamma, bn_beta, prelu_node,
           W_gcn, b_gcn, prelu_gcn):
    B, L, D = text_feature.shape
    vmem = pl.BlockSpec(memory_space=pltpu.VMEM)
    hbm = pl.BlockSpec(memory_space=pltpu.MemorySpace.HBM)
    return pl.pallas_call(
        _fused_kernel,
        in_specs=[vmem, hbm, vmem, vmem, vmem, vmem, vmem, vmem, vmem, vmem],
        out_specs=hbm,
        out_shape=jax.ShapeDtypeStruct((B, L, D), jnp.float32),
        scratch_shapes=[
            pltpu.VMEM((B, L, L), jnp.int32),
            pltpu.VMEM((B, L, D), jnp.float32),
            pltpu.SemaphoreType.DMA((B,)),
            pltpu.SemaphoreType.DMA((B,)),
        ],
    )(text_feature, adj, W_node,
      b_node.reshape(1, D), bn_gamma.reshape(1, D), bn_beta.reshape(1, D),
      prelu_node.reshape(1, 1), W_gcn, b_gcn.reshape(1, D),
      prelu_gcn.reshape(1, 1))
